# baseline (speedup 1.0000x reference)
import functools

import jax
import jax.numpy as jnp
from jax import lax
from jax.experimental import pallas as pl
from jax.experimental.pallas import tpu as pltpu
from jax.experimental.pallas import tpu_sc as plsc

N = 10000
NE = 320000
DH = 8
ACCW = 8
NDR = 640
NC, NS, L = 2, 16, 16
NW = NC * NS
EPW = NE // NW
GRPS = EPW // L
STRIPE = 1000
NST = N // STRIPE
ZR = 40
ECH = 2000
NCH = EPW // ECH
B = 80
NB = ECH // B

_HIGH = lax.Precision.HIGHEST


def _k1_body(x_ref, w1_ref, as_ref, ad_ref, wh_ref, s_ref, d_ref):
    wh = jnp.dot(x_ref[...], w1_ref[...], precision=_HIGH)
    wh_ref[...] = wh
    s_ref[...] = jnp.dot(wh, as_ref[...], precision=_HIGH)
    d_ref[...] = jnp.dot(wh, ad_ref[...], precision=_HIGH)


def _tc1(X, W1, a_src, a_dst):
    blk = 1000
    grid = N // blk
    return pl.pallas_call(
        _k1_body,
        grid=(grid,),
        in_specs=[
            pl.BlockSpec((blk, 128), lambda i: (i, 0)),
            pl.BlockSpec((128, DH), lambda i: (0, 0)),
            pl.BlockSpec((DH, 1), lambda i: (0, 0)),
            pl.BlockSpec((DH, 1), lambda i: (0, 0)),
        ],
        out_specs=[
            pl.BlockSpec((blk, DH), lambda i: (i, 0)),
            pl.BlockSpec((blk, 1), lambda i: (i, 0)),
            pl.BlockSpec((blk, 1), lambda i: (i, 0)),
        ],
        out_shape=[
            jax.ShapeDtypeStruct((N, DH), jnp.float32),
            jax.ShapeDtypeStruct((N, 1), jnp.float32),
            jax.ShapeDtypeStruct((N, 1), jnp.float32),
        ],
    )(X, W1, a_src, a_dst)


def _k2_body(acc_ref, den_ref, w2_ref, as_ref, ad_ref, h_ref, s_ref, d_ref):
    num = acc_ref[0] + acc_ref[1]
    den = den_ref[0] + den_ref[1] + 1e-16
    hv = num / den
    h = jnp.where(hv > 0, hv, jnp.exp(hv) - 1.0)
    h_ref[...] = h
    bs = jnp.dot(w2_ref[...], as_ref[...], precision=_HIGH)
    bd = jnp.dot(w2_ref[...], ad_ref[...], precision=_HIGH)
    s_ref[...] = jnp.dot(h, bs, precision=_HIGH)
    d_ref[...] = jnp.dot(h, bd, precision=_HIGH)


def _tc2(acc, den, W2, a_src, a_dst):
    blk = 1000
    grid = N // blk
    return pl.pallas_call(
        _k2_body,
        grid=(grid,),
        in_specs=[
            pl.BlockSpec((2, blk, ACCW), lambda i: (0, i, 0)),
            pl.BlockSpec((2, blk, 1), lambda i: (0, i, 0)),
            pl.BlockSpec((DH, 128), lambda i: (0, 0)),
            pl.BlockSpec((128, 1), lambda i: (0, 0)),
            pl.BlockSpec((128, 1), lambda i: (0, 0)),
        ],
        out_specs=[
            pl.BlockSpec((blk, DH), lambda i: (i, 0)),
            pl.BlockSpec((blk, 1), lambda i: (i, 0)),
            pl.BlockSpec((blk, 1), lambda i: (i, 0)),
        ],
        out_shape=[
            jax.ShapeDtypeStruct((N, DH), jnp.float32),
            jax.ShapeDtypeStruct((N, 1), jnp.float32),
            jax.ShapeDtypeStruct((N, 1), jnp.float32),
        ],
    )(acc, den, W2, a_src, a_dst)


def _k3_body(acc_ref, den_ref, w2_ref, o_ref):
    num = acc_ref[0] + acc_ref[1]
    den = den_ref[0] + den_ref[1] + 1e-16
    o_ref[...] = jnp.dot(num / den, w2_ref[...], precision=_HIGH)


def _tc3(acc, den, W2):
    blk = 1000
    grid = N // blk
    return pl.pallas_call(
        _k3_body,
        grid=(grid,),
        in_specs=[
            pl.BlockSpec((2, blk, ACCW), lambda i: (0, i, 0)),
            pl.BlockSpec((2, blk, 1), lambda i: (0, i, 0)),
            pl.BlockSpec((DH, 128), lambda i: (0, 0)),
        ],
        out_specs=pl.BlockSpec((blk, 128), lambda i: (i, 0)),
        out_shape=jax.ShapeDtypeStruct((N, 128), jnp.float32),
    )(acc, den, W2)


def _edge_body(src_hbm, dst_hbm, s_hbm, d_hbm, p_hbm, z8_hbm, z16_hbm,
               acc_hbm, den_hbm,
               src_v, dst_v, s_v, d_v, p_v, buf0, buf1, idx0, idx1,
               sem0, sem1, den_v, idxr_v, acc_sh, den_sh):
    c = lax.axis_index("c")
    t = lax.axis_index("s")
    wid = c * NS + t

    pltpu.sync_copy(s_hbm, s_v)
    pltpu.sync_copy(d_hbm, d_v)
    pltpu.sync_copy(p_hbm, p_v)
    pltpu.sync_copy(z16_hbm, den_v)

    @pl.when(t < NST)
    def _zero():
        pltpu.sync_copy(z8_hbm, acc_sh.at[pl.ds(t * STRIPE, STRIPE)])

    @pl.when(t == NST)
    def _zero_den():
        pltpu.sync_copy(z16_hbm, den_sh)

    lanes = lax.iota(jnp.int32, L)
    for k in range(NDR // L):
        idxr_v[pl.ds(k * L, L)] = lanes + jnp.int32(k * L)
    plsc.subcore_barrier()

    jcols = [jnp.full((L,), j, jnp.int32) for j in range(DH)]
    c15 = jnp.full((L,), 15, jnp.int32)
    c4 = jnp.full((L,), 4, jnp.int32)

    def fill_fire(bufk, idxk, semk, q):
        for sub in range(B // L):
            base = q * B + sub * L
            off = jnp.int32(sub * L)
            sv = src_v[pl.ds(base, L)]
            dv = dst_v[pl.ds(base, L)]
            se = plsc.load_gather(s_v, [sv])
            de = plsc.load_gather(d_v, [dv])
            e = se + de
            e = jnp.where(e >= 0, e, e * jnp.float32(0.2))
            w = jnp.exp(e)
            plsc.addupdate_scatter(
                den_v, [lax.shift_right_logical(dv, c4), dv & c15], w)
            for j in range(DH):
                colv = plsc.load_gather(p_v, [sv, jcols[j]])
                plsc.store_scatter(bufk, [lanes + off, jcols[j]], colv * w)
            idxk[pl.ds(sub * L, L)] = dv
        pltpu.async_copy(bufk, acc_sh.at[idxk], semk, add=True)

    def wait_k(bufk, idxk, semk):
        pltpu.make_async_copy(bufk, acc_sh.at[idxk], semk).wait()

    for ch in range(NCH):
        pltpu.sync_copy(src_hbm.at[pl.ds(wid * EPW + ch * ECH, ECH)], src_v)
        pltpu.sync_copy(dst_hbm.at[pl.ds(wid * EPW + ch * ECH, ECH)], dst_v)

        def pair(i, carry):
            @pl.when(i > 0)
            def _w0():
                wait_k(buf0, idx0, sem0)
            fill_fire(buf0, idx0, sem0, i * 2)

            @pl.when(i > 0)
            def _w1():
                wait_k(buf1, idx1, sem1)
            fill_fire(buf1, idx1, sem1, i * 2 + 1)
            return carry

        lax.fori_loop(0, NB // 2, pair, 0)
        wait_k(buf0, idx0, sem0)
        fill_fire(buf0, idx0, sem0, jnp.int32(NB - 1))
        wait_k(buf1, idx1, sem1)
        wait_k(buf0, idx0, sem0)

    pltpu.sync_copy(den_v, den_sh.at[idxr_v], add=True)
    plsc.subcore_barrier()

    @pl.when(t < NST)
    def _writeout():
        pltpu.sync_copy(acc_sh.at[pl.ds(t * STRIPE, STRIPE)],
                        acc_hbm.at[c, pl.ds(t * STRIPE, STRIPE)])

    @pl.when(t == NST)
    def _writeout_den():
        pltpu.sync_copy(den_sh, den_hbm.at[c])


def _edge_pass(src, dst, s, d, P, Z8, Z16):
    mesh = plsc.VectorSubcoreMesh(core_axis_name="c", subcore_axis_name="s")
    f = functools.partial(
        pl.kernel,
        mesh=mesh,
        compiler_params=pltpu.CompilerParams(
            needs_layout_passes=False, use_tc_tiling_on_sc=False),
        out_type=[
            jax.ShapeDtypeStruct((NC, N, ACCW), jnp.float32),
            jax.ShapeDtypeStruct((NC, NDR, L), jnp.float32),
        ],
        scratch_types=[
            pltpu.VMEM((ECH,), jnp.int32),
            pltpu.VMEM((ECH,), jnp.int32),
            pltpu.VMEM((N,), jnp.float32),
            pltpu.VMEM((N,), jnp.float32),
            pltpu.VMEM((N, DH), jnp.float32),
            pltpu.VMEM((B, ACCW), jnp.float32),
            pltpu.VMEM((B, ACCW), jnp.float32),
            pltpu.VMEM((B,), jnp.int32),
            pltpu.VMEM((B,), jnp.int32),
            pltpu.SemaphoreType.DMA,
            pltpu.SemaphoreType.DMA,
            pltpu.VMEM((NDR, L), jnp.float32),
            pltpu.VMEM((NDR,), jnp.int32),
            pltpu.VMEM_SHARED((N, ACCW), jnp.float32),
            pltpu.VMEM_SHARED((NDR, L), jnp.float32),
        ],
    )(_edge_body)
    return f(src, dst, s, d, P, Z8, Z16)


def kernel(V, E, X, W1, a1_src, a1_dst, W2, a2_src, a2_dst):
    del V
    E32 = E.astype(jnp.int32)
    src = E32[0]
    dst = E32[1]

    Z8 = jnp.zeros((STRIPE, ACCW), jnp.float32)
    Z16 = jnp.zeros((NDR, L), jnp.float32)

    def _den(dr):
        return dr.reshape(NC, NDR * L)[:, :N].reshape(NC, N, 1)

    wh1, s1, d1 = _tc1(X, W1, a1_src.reshape(DH, 1), a1_dst.reshape(DH, 1))
    acc1, den1 = _edge_pass(src, dst, s1.reshape(N), d1.reshape(N), wh1,
                            Z8, Z16)
    h, s2, d2 = _tc2(acc1, _den(den1), W2,
                     a2_src.reshape(128, 1), a2_dst.reshape(128, 1))
    acc2, den2 = _edge_pass(src, dst, s2.reshape(N), d2.reshape(N), h,
                            Z8, Z16)
    return _tc3(acc2, _den(den2), W2)


# speedup vs baseline: 1.0672x; 1.0672x over previous
import functools

import jax
import jax.numpy as jnp
from jax import lax
from jax.experimental import pallas as pl
from jax.experimental.pallas import tpu as pltpu
from jax.experimental.pallas import tpu_sc as plsc

N = 10000
NE = 320000
DH = 8
ACCW = 16
NC, NS, L = 2, 16, 16
NW = NC * NS
EPW = NE // NW
STRIPE = 1000
NST = N // STRIPE
ECH = 2000
NCH = EPW // ECH
B = 80
NB = ECH // B

_HIGH = lax.Precision.HIGHEST


def _k1_body(x_ref, w1_ref, as_ref, ad_ref, wh_ref, s_ref, d_ref):
    wh = jnp.dot(x_ref[...], w1_ref[...], precision=_HIGH)
    wh_ref[...] = wh
    s_ref[...] = jnp.dot(wh, as_ref[...], precision=_HIGH)
    d_ref[...] = jnp.dot(wh, ad_ref[...], precision=_HIGH)


def _tc1(X, W1, a_src, a_dst):
    blk = 1000
    grid = N // blk
    return pl.pallas_call(
        _k1_body,
        grid=(grid,),
        in_specs=[
            pl.BlockSpec((blk, 128), lambda i: (i, 0)),
            pl.BlockSpec((128, DH), lambda i: (0, 0)),
            pl.BlockSpec((DH, 1), lambda i: (0, 0)),
            pl.BlockSpec((DH, 1), lambda i: (0, 0)),
        ],
        out_specs=[
            pl.BlockSpec((blk, DH), lambda i: (i, 0)),
            pl.BlockSpec((blk, 1), lambda i: (i, 0)),
            pl.BlockSpec((blk, 1), lambda i: (i, 0)),
        ],
        out_shape=[
            jax.ShapeDtypeStruct((N, DH), jnp.float32),
            jax.ShapeDtypeStruct((N, 1), jnp.float32),
            jax.ShapeDtypeStruct((N, 1), jnp.float32),
        ],
    )(X, W1, a_src, a_dst)


def _k2_body(acc_ref, w2_ref, as_ref, ad_ref, h_ref, s_ref, d_ref):
    A = acc_ref[0] + acc_ref[1]
    num = A[:, 0:DH]
    den = A[:, DH:DH + 1] + 1e-16
    hv = num / den
    h = jnp.where(hv > 0, hv, jnp.exp(hv) - 1.0)
    h_ref[...] = h
    bs = jnp.dot(w2_ref[...], as_ref[...], precision=_HIGH)
    bd = jnp.dot(w2_ref[...], ad_ref[...], precision=_HIGH)
    s_ref[...] = jnp.dot(h, bs, precision=_HIGH)
    d_ref[...] = jnp.dot(h, bd, precision=_HIGH)


def _tc2(acc, W2, a_src, a_dst):
    blk = 1000
    grid = N // blk
    return pl.pallas_call(
        _k2_body,
        grid=(grid,),
        in_specs=[
            pl.BlockSpec((2, blk, ACCW), lambda i: (0, i, 0)),
            pl.BlockSpec((DH, 128), lambda i: (0, 0)),
            pl.BlockSpec((128, 1), lambda i: (0, 0)),
            pl.BlockSpec((128, 1), lambda i: (0, 0)),
        ],
        out_specs=[
            pl.BlockSpec((blk, DH), lambda i: (i, 0)),
            pl.BlockSpec((blk, 1), lambda i: (i, 0)),
            pl.BlockSpec((blk, 1), lambda i: (i, 0)),
        ],
        out_shape=[
            jax.ShapeDtypeStruct((N, DH), jnp.float32),
            jax.ShapeDtypeStruct((N, 1), jnp.float32),
            jax.ShapeDtypeStruct((N, 1), jnp.float32),
        ],
    )(acc, W2, a_src, a_dst)


def _k3_body(acc_ref, w2_ref, o_ref):
    A = acc_ref[0] + acc_ref[1]
    num = A[:, 0:DH]
    den = A[:, DH:DH + 1] + 1e-16
    o_ref[...] = jnp.dot(num / den, w2_ref[...], precision=_HIGH)


def _tc3(acc, W2):
    blk = 1000
    grid = N // blk
    return pl.pallas_call(
        _k3_body,
        grid=(grid,),
        in_specs=[
            pl.BlockSpec((2, blk, ACCW), lambda i: (0, i, 0)),
            pl.BlockSpec((DH, 128), lambda i: (0, 0)),
        ],
        out_specs=pl.BlockSpec((blk, 128), lambda i: (i, 0)),
        out_shape=jax.ShapeDtypeStruct((N, 128), jnp.float32),
    )(acc, W2)


def _edge_body(src_hbm, dst_hbm, s_hbm, d_hbm, p_hbm, z_hbm, acc_hbm,
               src_v, dst_v, s_v, d_v, p_v, buf0, buf1, idx0, idx1,
               sem0, sem1, acc_sh):
    c = lax.axis_index("c")
    t = lax.axis_index("s")
    wid = c * NS + t

    pltpu.sync_copy(s_hbm, s_v)
    pltpu.sync_copy(d_hbm, d_v)
    pltpu.sync_copy(p_hbm, p_v)

    zeros16 = jnp.zeros((L,), jnp.float32)
    for r in range(B):
        buf0[r, :] = zeros16
        buf1[r, :] = zeros16

    @pl.when(t < NST)
    def _zero():
        pltpu.sync_copy(z_hbm, acc_sh.at[pl.ds(t * STRIPE, STRIPE)])
    plsc.subcore_barrier()

    lanes = lax.iota(jnp.int32, L)
    col8 = jnp.full((L,), DH, jnp.int32)
    jcols = [jnp.full((L,), j, jnp.int32) for j in range(DH)]

    def fill_fire(bufk, idxk, semk, q):
        for sub in range(B // L):
            base = q * B + sub * L
            off = jnp.int32(sub * L)
            sv = src_v[pl.ds(base, L)]
            dv = dst_v[pl.ds(base, L)]
            se = plsc.load_gather(s_v, [sv])
            de = plsc.load_gather(d_v, [dv])
            e = se + de
            e = jnp.where(e >= 0, e, e * jnp.float32(0.2))
            w = jnp.exp(e)
            plsc.store_scatter(bufk, [lanes + off, col8], w)
            for j in range(DH):
                colv = plsc.load_gather(p_v, [sv, jcols[j]])
                plsc.store_scatter(bufk, [lanes + off, jcols[j]], colv * w)
            idxk[pl.ds(sub * L, L)] = dv
        pltpu.async_copy(bufk, acc_sh.at[idxk], semk, add=True)

    def wait_k(bufk, idxk, semk):
        pltpu.make_async_copy(bufk, acc_sh.at[idxk], semk).wait()

    for ch in range(NCH):
        pltpu.sync_copy(src_hbm.at[pl.ds(wid * EPW + ch * ECH, ECH)], src_v)
        pltpu.sync_copy(dst_hbm.at[pl.ds(wid * EPW + ch * ECH, ECH)], dst_v)

        def pair(i, carry):
            @pl.when(i > 0)
            def _w0():
                wait_k(buf0, idx0, sem0)
            fill_fire(buf0, idx0, sem0, i * 2)

            @pl.when(i > 0)
            def _w1():
                wait_k(buf1, idx1, sem1)
            fill_fire(buf1, idx1, sem1, i * 2 + 1)
            return carry

        lax.fori_loop(0, NB // 2, pair, 0)
        wait_k(buf0, idx0, sem0)
        fill_fire(buf0, idx0, sem0, jnp.int32(NB - 1))
        wait_k(buf1, idx1, sem1)
        wait_k(buf0, idx0, sem0)
    plsc.subcore_barrier()

    @pl.when(t < NST)
    def _writeout():
        pltpu.sync_copy(acc_sh.at[pl.ds(t * STRIPE, STRIPE)],
                        acc_hbm.at[c, pl.ds(t * STRIPE, STRIPE)])


def _edge_pass(src, dst, s, d, P, Z):
    mesh = plsc.VectorSubcoreMesh(core_axis_name="c", subcore_axis_name="s")
    f = functools.partial(
        pl.kernel,
        mesh=mesh,
        compiler_params=pltpu.CompilerParams(
            needs_layout_passes=False, use_tc_tiling_on_sc=False),
        out_type=jax.ShapeDtypeStruct((NC, N, ACCW), jnp.float32),
        scratch_types=[
            pltpu.VMEM((ECH,), jnp.int32),
            pltpu.VMEM((ECH,), jnp.int32),
            pltpu.VMEM((N,), jnp.float32),
            pltpu.VMEM((N,), jnp.float32),
            pltpu.VMEM((N, DH), jnp.float32),
            pltpu.VMEM((B, ACCW), jnp.float32),
            pltpu.VMEM((B, ACCW), jnp.float32),
            pltpu.VMEM((B,), jnp.int32),
            pltpu.VMEM((B,), jnp.int32),
            pltpu.SemaphoreType.DMA,
            pltpu.SemaphoreType.DMA,
            pltpu.VMEM_SHARED((N, ACCW), jnp.float32),
        ],
    )(_edge_body)
    return f(src, dst, s, d, P, Z)


def kernel(V, E, X, W1, a1_src, a1_dst, W2, a2_src, a2_dst):
    del V
    E32 = E.astype(jnp.int32)
    src = E32[0]
    dst = E32[1]

    Z = jnp.zeros((STRIPE, ACCW), jnp.float32)
    wh1, s1, d1 = _tc1(X, W1, a1_src.reshape(DH, 1), a1_dst.reshape(DH, 1))
    acc1 = _edge_pass(src, dst, s1.reshape(N), d1.reshape(N), wh1, Z)
    h, s2, d2 = _tc2(acc1, W2, a2_src.reshape(128, 1), a2_dst.reshape(128, 1))
    acc2 = _edge_pass(src, dst, s2.reshape(N), d2.reshape(N), h, Z)
    return _tc3(acc2, W2)
